# baseline (speedup 1.0000x reference)
import functools

import jax
import jax.numpy as jnp
from jax import lax
from jax.experimental import pallas as pl
from jax.experimental.pallas import tpu as pltpu
from jax.experimental.pallas import tpu_sc as plsc

T_BLOCK = 1024
E = 64
TOP_K = 8
N_GROUPS = 8
GROUP = E // N_GROUPS
ROUTED_SCALING_FACTOR = 2.5
N_WORKERS = 32
LANES = 16
CHUNK = 256


def _partner(x, k):
    row = jax.lax.broadcasted_iota(jnp.int32, x.shape, 0)
    bit = (row & k) != 0
    n = x.shape[0]
    return jnp.where(bit, pltpu.roll(x, k, axis=0),
                     pltpu.roll(x, n - k, axis=0))


def _mask_tile(st, bias_col):
    row = jax.lax.broadcasted_iota(jnp.int32, st.shape, 0)
    scores = 1.0 / (1.0 + jnp.exp(-st))
    sb = scores + bias_col

    m1 = sb
    m2 = jnp.full_like(sb, -jnp.inf)
    for k in (1, 2, 4):
        p1 = _partner(m1, k)
        p2 = _partner(m2, k)
        m1, m2 = (jnp.maximum(m1, p1),
                  jnp.maximum(jnp.minimum(m1, p1), jnp.maximum(m2, p2)))
    gs = m1 + m2

    beat_cnt = jnp.zeros_like(row)
    for d in range(1, N_GROUPS):
        other = pltpu.roll(gs, GROUP * d, axis=0)
        tie_lower = (row // GROUP) >= d
        beats = (other > gs) | ((other == gs) & tie_lower)
        beat_cnt = beat_cnt + beats.astype(jnp.int32)
    masked = jnp.where(beat_cnt < 4, sb, 0.0)
    return masked, scores


def _tc_body(xa_ref, xb_ref, w_ref, b_ref, mout_ref, sout_ref, scratch_ref):
    i = pl.program_id(0)
    n = pl.num_programs(0)

    @pl.when(i < n - 1)
    def _produce():
        dn = (((1,), (0,)), ((), ()))
        s = jax.lax.dot_general(
            xa_ref[...], w_ref[0], dn,
            preferred_element_type=jnp.float32,
            precision=jax.lax.Precision.DEFAULT)
        s = s + jax.lax.dot_general(
            xb_ref[...], w_ref[1], dn,
            preferred_element_type=jnp.float32,
            precision=jax.lax.Precision.DEFAULT)
        scratch_ref[i % 2] = s.T

    @pl.when(i > 0)
    def _consume():
        masked, scores = _mask_tile(scratch_ref[(i - 1) % 2], b_ref[...])
        for k in range(masked.shape[1] // CHUNK):
            mout_ref[k] = masked[:, k * CHUNK:(k + 1) * CHUNK]
            sout_ref[k] = scores[:, k * CHUNK:(k + 1) * CHUNK]


def _tc_stage(x_TD, kernel_DE, bias_E, chunk_idx=0, n_chunks=1):
    T, D = x_TD.shape
    tch = T // n_chunks
    bt = T_BLOCK if tch % T_BLOCK == 0 else tch
    n_blocks = tch // bt
    blk0 = chunk_idx * n_blocks
    n_slabs_blk = bt // CHUNK
    h = D // 2
    x32 = x_TD.astype(jnp.float32)
    w_2hE = kernel_DE.reshape(2, h, E)

    def x_map(i):
        return (blk0 + jnp.minimum(i, n_blocks - 1), 0)

    def x_map_b(i):
        return (blk0 + jnp.minimum(i, n_blocks - 1), 1)

    return pl.pallas_call(
        _tc_body,
        grid=(n_blocks + 1,),
        in_specs=[
            pl.BlockSpec((bt, h), x_map),
            pl.BlockSpec((bt, h), x_map_b),
            pl.BlockSpec((2, h, E), lambda i: (0, 0, 0)),
            pl.BlockSpec((E, 1), lambda i: (0, 0)),
        ],
        out_specs=[
            pl.BlockSpec((n_slabs_blk, E, CHUNK),
                         lambda i: (jnp.maximum(i - 1, 0), 0, 0)),
            pl.BlockSpec((n_slabs_blk, E, CHUNK),
                         lambda i: (jnp.maximum(i - 1, 0), 0, 0)),
        ],
        out_shape=[
            jax.ShapeDtypeStruct((tch // CHUNK, E, CHUNK), jnp.float32),
            jax.ShapeDtypeStruct((tch // CHUNK, E, CHUNK), jnp.float32),
        ],
        scratch_shapes=[pltpu.VMEM((2, E, bt), jnp.float32)],
        compiler_params=pltpu.CompilerParams(
            dimension_semantics=("arbitrary",)),
    )(x32, x32, w_2hE, bias_E.reshape(E, 1))


def _sc_route_body(n_slabs_w, mv_hbm, s_hbm, wout_hbm, iout_hbm,
                   mv_v0, mv_v1, s_v0, s_v1, wv0, wv1, iv0, iv1,
                   sem_m0, sem_m1, sem_s0, sem_s1, sem_o0, sem_o1):
    wid = lax.axis_index("s") * 2 + lax.axis_index("c")
    lanes = lax.iota(jnp.int32, LANES)
    neg_inf = jnp.full((LANES,), -jnp.inf, jnp.float32)
    in_sz = E * CHUNK
    out_sz = TOP_K * CHUNK
    mv_bufs, s_bufs = (mv_v0, mv_v1), (s_v0, s_v1)
    w_bufs, i_bufs = (wv0, wv1), (iv0, iv1)
    in_sems = ((sem_m0, sem_s0), (sem_m1, sem_s1))
    out_sems = (sem_o0, sem_o1)

    def make_tile_body(mv_v, s_v, wv, iv):
        def tile_body(ti, carry):
            tj = ti * LANES
            cols = tj + lanes
            gmax = []
            for g in range(N_GROUPS):
                m = mv_v[pl.ds(g * GROUP * CHUNK + tj, LANES)]
                for r in range(1, GROUP):
                    m = jnp.maximum(
                        m, mv_v[pl.ds((g * GROUP + r) * CHUNK + tj, LANES)])
                gmax.append(m)

            ws, idxs = [], []
            for _ in range(TOP_K):
                m = gmax[0]
                for g in range(1, N_GROUPS):
                    m = jnp.maximum(m, gmax[g])
                grp = jnp.full((LANES,), N_GROUPS - 1, jnp.int32)
                for g in range(N_GROUPS - 2, -1, -1):
                    grp = jnp.where(gmax[g] == m, g, grp)
                rowvals = [
                    plsc.load_gather(mv_v, [(grp * GROUP + r) * CHUNK + cols])
                    for r in range(GROUP)]
                rr = jnp.full((LANES,), GROUP - 1, jnp.int32)
                for r in range(GROUP - 2, -1, -1):
                    rr = jnp.where(rowvals[r] == m, r, rr)
                idx = grp * GROUP + rr
                ws.append(plsc.load_gather(s_v, [idx * CHUNK + cols]))
                idxs.append(idx)
                plsc.store_scatter(mv_v, [idx * CHUNK + cols], neg_inf)
                nm = jnp.where(rr == 0, neg_inf, rowvals[0])
                for r in range(1, GROUP):
                    nm = jnp.maximum(
                        nm, jnp.where(rr == r, neg_inf, rowvals[r]))
                for g in range(N_GROUPS):
                    gmax[g] = jnp.where(grp == g, nm, gmax[g])

            wsum = ws[0]
            for j in range(1, TOP_K):
                wsum = wsum + ws[j]
            wsum = wsum + 1e-20
            for j in range(TOP_K):
                wv[pl.ds(j * CHUNK + tj, LANES)] = (
                    (ws[j] / wsum) * ROUTED_SCALING_FACTOR)
                iv[pl.ds(j * CHUNK + tj, LANES)] = idxs[j]
            return carry
        return tile_body

    def start_in(si, b):
        slab = wid * n_slabs_w + si
        src = mv_hbm.at[pl.ds(slab * in_sz, in_sz)]
        h_m = pltpu.make_async_copy(src, mv_bufs[b], in_sems[b][0])
        h_m.start()
        src = s_hbm.at[pl.ds(slab * in_sz, in_sz)]
        h_s = pltpu.make_async_copy(src, s_bufs[b], in_sems[b][1])
        h_s.start()
        return h_m, h_s

    out_handles = []
    pending = start_in(0, 0)
    for si in range(n_slabs_w):
        b = si % 2
        slab = wid * n_slabs_w + si
        pending[0].wait()
        pending[1].wait()
        if si + 1 < n_slabs_w:
            pending = start_in(si + 1, (si + 1) % 2)
        if len(out_handles) >= 2:
            for h in out_handles.pop(0):
                h.wait()
        lax.fori_loop(0, CHUNK // LANES,
                      make_tile_body(mv_bufs[b], s_bufs[b],
                                     w_bufs[b], i_bufs[b]), 0)
        hw = pltpu.make_async_copy(
            w_bufs[b], wout_hbm.at[pl.ds(slab * out_sz, out_sz)], out_sems[b])
        hw.start()
        hi = pltpu.make_async_copy(
            i_bufs[b], iout_hbm.at[pl.ds(slab * out_sz, out_sz)], out_sems[b])
        hi.start()
        out_handles.append((hw, hi))
    for hs in out_handles:
        for h in hs:
            h.wait()


def _sc_route(masked_slabs, scores_slabs):
    n_slabs = masked_slabs.shape[0]
    T = n_slabs * CHUNK
    n_slabs_w = n_slabs // N_WORKERS
    mesh = plsc.VectorSubcoreMesh(core_axis_name="c", subcore_axis_name="s")
    return pl.kernel(
        functools.partial(_sc_route_body, n_slabs_w),
        mesh=mesh,
        out_type=[
            jax.ShapeDtypeStruct((T * TOP_K,), jnp.float32),
            jax.ShapeDtypeStruct((T * TOP_K,), jnp.int32),
        ],
        scratch_types=[
            pltpu.VMEM((E * CHUNK,), jnp.float32),
            pltpu.VMEM((E * CHUNK,), jnp.float32),
            pltpu.VMEM((E * CHUNK,), jnp.float32),
            pltpu.VMEM((E * CHUNK,), jnp.float32),
            pltpu.VMEM((TOP_K * CHUNK,), jnp.float32),
            pltpu.VMEM((TOP_K * CHUNK,), jnp.float32),
            pltpu.VMEM((TOP_K * CHUNK,), jnp.int32),
            pltpu.VMEM((TOP_K * CHUNK,), jnp.int32),
            pltpu.SemaphoreType.DMA,
            pltpu.SemaphoreType.DMA,
            pltpu.SemaphoreType.DMA,
            pltpu.SemaphoreType.DMA,
            pltpu.SemaphoreType.DMA,
            pltpu.SemaphoreType.DMA,
        ],
        compiler_params=pltpu.CompilerParams(use_tc_tiling_on_sc=False,
                                             needs_layout_passes=False),
    )(masked_slabs.reshape(-1), scores_slabs.reshape(-1))


N_MACRO_CHUNKS = 1


@jax.jit
def kernel(x_TD, kernel_DE, bias_E):
    T = x_TD.shape[0]
    nch = N_MACRO_CHUNKS if T % (N_MACRO_CHUNKS * N_WORKERS * CHUNK) == 0 else 1
    tc = T // nch
    outs = []
    for c in range(nch):
        masked_slabs, scores_slabs = _tc_stage(
            x_TD, kernel_DE, bias_E, chunk_idx=c, n_chunks=nch)
        outs.append(_sc_route(masked_slabs, scores_slabs))
    ws = [wf.reshape(tc // CHUNK, TOP_K, CHUNK).transpose(0, 2, 1)
          for wf, _ in outs]
    idxs = [if_.reshape(tc // CHUNK, TOP_K, CHUNK).transpose(0, 2, 1)
            for _, if_ in outs]
    w = jnp.concatenate(ws, axis=0).reshape(T, TOP_K)
    idx = jnp.concatenate(idxs, axis=0).reshape(T, TOP_K)
    return (w, idx)


# speedup vs baseline: 1.0188x; 1.0188x over previous
import functools

import jax
import jax.numpy as jnp
from jax import lax
from jax.experimental import pallas as pl
from jax.experimental.pallas import tpu as pltpu
from jax.experimental.pallas import tpu_sc as plsc

T_BLOCK = 1024
E = 64
TOP_K = 8
N_GROUPS = 8
GROUP = E // N_GROUPS
ROUTED_SCALING_FACTOR = 2.5
N_WORKERS = 32
LANES = 16
CHUNK = 256


def _partner(x, k):
    row = jax.lax.broadcasted_iota(jnp.int32, x.shape, 0)
    bit = (row & k) != 0
    n = x.shape[0]
    return jnp.where(bit, pltpu.roll(x, k, axis=0),
                     pltpu.roll(x, n - k, axis=0))


def _mask_tile(st, bias_col):
    row = jax.lax.broadcasted_iota(jnp.int32, st.shape, 0)
    scores = 1.0 / (1.0 + jnp.exp(-st))
    sb = scores + bias_col

    m1 = sb
    m2 = jnp.full_like(sb, -jnp.inf)
    for k in (1, 2, 4):
        p1 = _partner(m1, k)
        p2 = _partner(m2, k)
        m1, m2 = (jnp.maximum(m1, p1),
                  jnp.maximum(jnp.minimum(m1, p1), jnp.maximum(m2, p2)))
    gs = m1 + m2

    beat_cnt = jnp.zeros_like(row)
    for d in range(1, N_GROUPS):
        other = pltpu.roll(gs, GROUP * d, axis=0)
        tie_lower = (row // GROUP) >= d
        beats = (other > gs) | ((other == gs) & tie_lower)
        beat_cnt = beat_cnt + beats.astype(jnp.int32)
    masked = jnp.where(beat_cnt < 4, sb, 0.0)
    return masked, scores


def _tc_body(xa_ref, xb_ref, w_ref, b_ref, mout_ref, sout_ref, scratch_ref):
    i = pl.program_id(0)
    n = pl.num_programs(0)

    @pl.when(i < n - 1)
    def _produce():
        dn = (((1,), (0,)), ((), ()))
        s = jax.lax.dot_general(
            xa_ref[...], w_ref[0], dn,
            preferred_element_type=jnp.float32,
            precision=jax.lax.Precision.DEFAULT)
        s = s + jax.lax.dot_general(
            xb_ref[...], w_ref[1], dn,
            preferred_element_type=jnp.float32,
            precision=jax.lax.Precision.DEFAULT)
        scratch_ref[i % 2] = s.T

    @pl.when(i > 0)
    def _consume():
        masked, scores = _mask_tile(scratch_ref[(i - 1) % 2], b_ref[...])
        for k in range(masked.shape[1] // CHUNK):
            mout_ref[k] = masked[:, k * CHUNK:(k + 1) * CHUNK]
            sout_ref[k] = scores[:, k * CHUNK:(k + 1) * CHUNK]


def _tc_stage(x_TD, kernel_DE, bias_E, chunk_idx=0, n_chunks=1):
    T, D = x_TD.shape
    tch = T // n_chunks
    bt = T_BLOCK if tch % T_BLOCK == 0 else tch
    n_blocks = tch // bt
    blk0 = chunk_idx * n_blocks
    n_slabs_blk = bt // CHUNK
    h = D // 2
    x32 = x_TD.astype(jnp.float32)
    w_2hE = kernel_DE.reshape(2, h, E)

    def x_map(i):
        return (blk0 + jnp.minimum(i, n_blocks - 1), 0)

    def x_map_b(i):
        return (blk0 + jnp.minimum(i, n_blocks - 1), 1)

    return pl.pallas_call(
        _tc_body,
        grid=(n_blocks + 1,),
        in_specs=[
            pl.BlockSpec((bt, h), x_map),
            pl.BlockSpec((bt, h), x_map_b),
            pl.BlockSpec((2, h, E), lambda i: (0, 0, 0)),
            pl.BlockSpec((E, 1), lambda i: (0, 0)),
        ],
        out_specs=[
            pl.BlockSpec((n_slabs_blk, E, CHUNK),
                         lambda i: (jnp.maximum(i - 1, 0), 0, 0)),
            pl.BlockSpec((n_slabs_blk, E, CHUNK),
                         lambda i: (jnp.maximum(i - 1, 0), 0, 0)),
        ],
        out_shape=[
            jax.ShapeDtypeStruct((tch // CHUNK, E, CHUNK), jnp.float32),
            jax.ShapeDtypeStruct((tch // CHUNK, E, CHUNK), jnp.float32),
        ],
        scratch_shapes=[pltpu.VMEM((2, E, bt), jnp.float32)],
        compiler_params=pltpu.CompilerParams(
            dimension_semantics=("arbitrary",)),
    )(x32, x32, w_2hE, bias_E.reshape(E, 1))


def _sc_route_body(n_slabs_w, mv_hbm, s_hbm, wout_hbm, iout_hbm,
                   mv_v0, mv_v1, s_v0, s_v1, wv0, wv1, iv0, iv1,
                   sem_m0, sem_m1, sem_s0, sem_s1, sem_o0, sem_o1):
    wid = lax.axis_index("s") * 2 + lax.axis_index("c")
    lanes = lax.iota(jnp.int32, LANES)
    neg_inf = jnp.full((LANES,), -jnp.inf, jnp.float32)
    in_sz = E * CHUNK
    out_sz = TOP_K * CHUNK
    mv_bufs, s_bufs = (mv_v0, mv_v1), (s_v0, s_v1)
    w_bufs, i_bufs = (wv0, wv1), (iv0, iv1)
    in_sems = ((sem_m0, sem_s0), (sem_m1, sem_s1))
    out_sems = (sem_o0, sem_o1)

    def make_tile_body(mv_v, s_v, wv, iv):
        def tile_body(ti, carry):
            tj = ti * LANES
            cols = tj + lanes
            gmax = []
            for g in range(N_GROUPS):
                m = mv_v[pl.ds(g * GROUP * CHUNK + tj, LANES)]
                for r in range(1, GROUP):
                    m = jnp.maximum(
                        m, mv_v[pl.ds((g * GROUP + r) * CHUNK + tj, LANES)])
                gmax.append(m)

            ws, idxs = [], []
            for _ in range(TOP_K):
                m = gmax[0]
                for g in range(1, N_GROUPS):
                    m = jnp.maximum(m, gmax[g])
                grp = jnp.full((LANES,), N_GROUPS - 1, jnp.int32)
                for g in range(N_GROUPS - 2, -1, -1):
                    grp = jnp.where(gmax[g] == m, g, grp)
                rowvals = [
                    plsc.load_gather(mv_v, [(grp * GROUP + r) * CHUNK + cols])
                    for r in range(GROUP)]
                rr = jnp.full((LANES,), GROUP - 1, jnp.int32)
                for r in range(GROUP - 2, -1, -1):
                    rr = jnp.where(rowvals[r] == m, r, rr)
                idx = grp * GROUP + rr
                ws.append(plsc.load_gather(s_v, [idx * CHUNK + cols]))
                idxs.append(idx)
                plsc.store_scatter(mv_v, [idx * CHUNK + cols], neg_inf)
                nm = jnp.where(rr == 0, neg_inf, rowvals[0])
                for r in range(1, GROUP):
                    nm = jnp.maximum(
                        nm, jnp.where(rr == r, neg_inf, rowvals[r]))
                for g in range(N_GROUPS):
                    gmax[g] = jnp.where(grp == g, nm, gmax[g])

            wsum = ws[0]
            for j in range(1, TOP_K):
                wsum = wsum + ws[j]
            wsum = wsum + 1e-20
            for j in range(TOP_K):
                wv[pl.ds(j * CHUNK + tj, LANES)] = (
                    (ws[j] / wsum) * ROUTED_SCALING_FACTOR)
                iv[pl.ds(j * CHUNK + tj, LANES)] = idxs[j]
            return carry
        return tile_body

    def start_in(si, b):
        slab = wid * n_slabs_w + si
        src = mv_hbm.at[pl.ds(slab * in_sz, in_sz)]
        h_m = pltpu.make_async_copy(src, mv_bufs[b], in_sems[b][0])
        h_m.start()
        src = s_hbm.at[pl.ds(slab * in_sz, in_sz)]
        h_s = pltpu.make_async_copy(src, s_bufs[b], in_sems[b][1])
        h_s.start()
        return h_m, h_s

    out_handles = []
    pending = start_in(0, 0)
    for si in range(n_slabs_w):
        b = si % 2
        slab = wid * n_slabs_w + si
        pending[0].wait()
        pending[1].wait()
        if si + 1 < n_slabs_w:
            pending = start_in(si + 1, (si + 1) % 2)
        if len(out_handles) >= 2:
            for h in out_handles.pop(0):
                h.wait()
        lax.fori_loop(0, CHUNK // LANES,
                      make_tile_body(mv_bufs[b], s_bufs[b],
                                     w_bufs[b], i_bufs[b]), 0)
        hw = pltpu.make_async_copy(
            w_bufs[b], wout_hbm.at[pl.ds(slab * out_sz, out_sz)], out_sems[b])
        hw.start()
        hi = pltpu.make_async_copy(
            i_bufs[b], iout_hbm.at[pl.ds(slab * out_sz, out_sz)], out_sems[b])
        hi.start()
        out_handles.append((hw, hi))
    for hs in out_handles:
        for h in hs:
            h.wait()


def _sc_route(masked_slabs, scores_slabs):
    n_slabs = masked_slabs.shape[0]
    T = n_slabs * CHUNK
    n_slabs_w = n_slabs // N_WORKERS
    mesh = plsc.VectorSubcoreMesh(core_axis_name="c", subcore_axis_name="s")
    return pl.kernel(
        functools.partial(_sc_route_body, n_slabs_w),
        mesh=mesh,
        out_type=[
            jax.ShapeDtypeStruct((T * TOP_K,), jnp.float32),
            jax.ShapeDtypeStruct((T * TOP_K,), jnp.int32),
        ],
        scratch_types=[
            pltpu.VMEM((E * CHUNK,), jnp.float32),
            pltpu.VMEM((E * CHUNK,), jnp.float32),
            pltpu.VMEM((E * CHUNK,), jnp.float32),
            pltpu.VMEM((E * CHUNK,), jnp.float32),
            pltpu.VMEM((TOP_K * CHUNK,), jnp.float32),
            pltpu.VMEM((TOP_K * CHUNK,), jnp.float32),
            pltpu.VMEM((TOP_K * CHUNK,), jnp.int32),
            pltpu.VMEM((TOP_K * CHUNK,), jnp.int32),
            pltpu.SemaphoreType.DMA,
            pltpu.SemaphoreType.DMA,
            pltpu.SemaphoreType.DMA,
            pltpu.SemaphoreType.DMA,
            pltpu.SemaphoreType.DMA,
            pltpu.SemaphoreType.DMA,
        ],
        compiler_params=pltpu.CompilerParams(use_tc_tiling_on_sc=False,
                                             needs_layout_passes=False),
    )(masked_slabs.reshape(-1), scores_slabs.reshape(-1))


N_MACRO_CHUNKS = 2


@jax.jit
def kernel(x_TD, kernel_DE, bias_E):
    T = x_TD.shape[0]
    nch = N_MACRO_CHUNKS if T % (N_MACRO_CHUNKS * N_WORKERS * CHUNK) == 0 else 1
    tc = T // nch
    slabs = [_tc_stage(x_TD, kernel_DE, bias_E, chunk_idx=c, n_chunks=nch)
             for c in range(nch)]
    outs = [_sc_route(m, s) for m, s in slabs]
    ws = [wf.reshape(tc // CHUNK, TOP_K, CHUNK).transpose(0, 2, 1)
          for wf, _ in outs]
    idxs = [if_.reshape(tc // CHUNK, TOP_K, CHUNK).transpose(0, 2, 1)
            for _, if_ in outs]
    w = jnp.concatenate(ws, axis=0).reshape(T, TOP_K)
    idx = jnp.concatenate(idxs, axis=0).reshape(T, TOP_K)
    return (w, idx)


# speedup vs baseline: 1.0653x; 1.0456x over previous
import functools

import jax
import jax.numpy as jnp
from jax import lax
from jax.experimental import pallas as pl
from jax.experimental.pallas import tpu as pltpu
from jax.experimental.pallas import tpu_sc as plsc

T_BLOCK = 1024
E = 64
TOP_K = 8
N_GROUPS = 8
GROUP = E // N_GROUPS
ROUTED_SCALING_FACTOR = 2.5
N_WORKERS = 32
LANES = 16
CHUNK = 256


def _partner(x, k):
    row = jax.lax.broadcasted_iota(jnp.int32, x.shape, 0)
    bit = (row & k) != 0
    n = x.shape[0]
    return jnp.where(bit, pltpu.roll(x, k, axis=0),
                     pltpu.roll(x, n - k, axis=0))


def _mask_tile(st, bias_col):
    row = jax.lax.broadcasted_iota(jnp.int32, st.shape, 0)
    scores = 1.0 / (1.0 + jnp.exp(-st))
    sb = scores + bias_col

    m1 = sb
    m2 = jnp.full_like(sb, -jnp.inf)
    for k in (1, 2, 4):
        p1 = _partner(m1, k)
        p2 = _partner(m2, k)
        m1, m2 = (jnp.maximum(m1, p1),
                  jnp.maximum(jnp.minimum(m1, p1), jnp.maximum(m2, p2)))
    gs = m1 + m2

    beat_cnt = jnp.zeros_like(row)
    for d in range(1, N_GROUPS):
        other = pltpu.roll(gs, GROUP * d, axis=0)
        tie_lower = (row // GROUP) >= d
        beats = (other > gs) | ((other == gs) & tie_lower)
        beat_cnt = beat_cnt + beats.astype(jnp.int32)
    masked = jnp.where(beat_cnt < 4, sb, 0.0)

    work = masked
    idx_rows = []
    for _ in range(TOP_K):
        m = jnp.max(work, axis=0, keepdims=True)
        idx = jnp.min(jnp.where(work == m, row, E), axis=0, keepdims=True)
        onehot = row == idx
        idx_rows.append(idx)
        work = jnp.where(onehot, -jnp.inf, work)
    indices = jnp.concatenate(idx_rows, axis=0).astype(jnp.int32)
    return indices, scores


def _tc_body(xa_ref, xb_ref, w_ref, b_ref, mout_ref, sout_ref, scratch_ref):
    i = pl.program_id(0)
    n = pl.num_programs(0)

    @pl.when(i < n - 1)
    def _produce():
        dn = (((1,), (0,)), ((), ()))
        s = jax.lax.dot_general(
            xa_ref[...], w_ref[0], dn,
            preferred_element_type=jnp.float32,
            precision=jax.lax.Precision.DEFAULT)
        s = s + jax.lax.dot_general(
            xb_ref[...], w_ref[1], dn,
            preferred_element_type=jnp.float32,
            precision=jax.lax.Precision.DEFAULT)
        scratch_ref[i % 2] = s.T

    @pl.when(i > 0)
    def _consume():
        indices, scores = _mask_tile(scratch_ref[(i - 1) % 2], b_ref[...])
        for k in range(scores.shape[1] // CHUNK):
            mout_ref[k] = indices[:, k * CHUNK:(k + 1) * CHUNK]
            sout_ref[k] = scores[:, k * CHUNK:(k + 1) * CHUNK]


def _tc_stage(x_TD, kernel_DE, bias_E, chunk_idx=0, n_chunks=1):
    T, D = x_TD.shape
    tch = T // n_chunks
    bt = T_BLOCK if tch % T_BLOCK == 0 else tch
    n_blocks = tch // bt
    blk0 = chunk_idx * n_blocks
    n_slabs_blk = bt // CHUNK
    h = D // 2
    x32 = x_TD.astype(jnp.float32)
    w_2hE = kernel_DE.reshape(2, h, E)

    def x_map(i):
        return (blk0 + jnp.minimum(i, n_blocks - 1), 0)

    def x_map_b(i):
        return (blk0 + jnp.minimum(i, n_blocks - 1), 1)

    return pl.pallas_call(
        _tc_body,
        grid=(n_blocks + 1,),
        in_specs=[
            pl.BlockSpec((bt, h), x_map),
            pl.BlockSpec((bt, h), x_map_b),
            pl.BlockSpec((2, h, E), lambda i: (0, 0, 0)),
            pl.BlockSpec((E, 1), lambda i: (0, 0)),
        ],
        out_specs=[
            pl.BlockSpec((n_slabs_blk, TOP_K, CHUNK),
                         lambda i: (jnp.maximum(i - 1, 0), 0, 0)),
            pl.BlockSpec((n_slabs_blk, E, CHUNK),
                         lambda i: (jnp.maximum(i - 1, 0), 0, 0)),
        ],
        out_shape=[
            jax.ShapeDtypeStruct((tch // CHUNK, TOP_K, CHUNK), jnp.int32),
            jax.ShapeDtypeStruct((tch // CHUNK, E, CHUNK), jnp.float32),
        ],
        scratch_shapes=[pltpu.VMEM((2, E, bt), jnp.float32)],
        compiler_params=pltpu.CompilerParams(
            dimension_semantics=("arbitrary",)),
    )(x32, x32, w_2hE, bias_E.reshape(E, 1))


def _sc_route_body(n_slabs_w, mv_hbm, s_hbm, wout_hbm, iout_hbm,
                   mv_v0, mv_v1, s_v0, s_v1, wv0, wv1, iv0, iv1,
                   sem_m0, sem_m1, sem_s0, sem_s1, sem_o0, sem_o1):
    wid = lax.axis_index("s") * 2 + lax.axis_index("c")
    lanes = lax.iota(jnp.int32, LANES)
    in_sz_i = TOP_K * CHUNK
    in_sz_s = E * CHUNK
    out_sz = TOP_K * CHUNK
    mv_bufs, s_bufs = (mv_v0, mv_v1), (s_v0, s_v1)
    w_bufs, i_bufs = (wv0, wv1), (iv0, iv1)
    in_sems = ((sem_m0, sem_s0), (sem_m1, sem_s1))
    out_sems = (sem_o0, sem_o1)

    def make_tile_body(mv_v, s_v, wv, iv):
        def tile_body(ti, carry):
            tj = ti * LANES
            cols = tj + lanes
            ws, idxs = [], []
            for j in range(TOP_K):
                iv_j = mv_v[pl.ds(j * CHUNK + tj, LANES)]
                ws.append(plsc.load_gather(s_v, [iv_j * CHUNK + cols]))
                idxs.append(iv_j)
            wsum = ws[0]
            for j in range(1, TOP_K):
                wsum = wsum + ws[j]
            wsum = wsum + 1e-20
            for j in range(TOP_K):
                wv[pl.ds(j * CHUNK + tj, LANES)] = (
                    (ws[j] / wsum) * ROUTED_SCALING_FACTOR)
                iv[pl.ds(j * CHUNK + tj, LANES)] = idxs[j]
            return carry
        return tile_body

    def start_in(si, b):
        slab = wid * n_slabs_w + si
        src = mv_hbm.at[pl.ds(slab * in_sz_i, in_sz_i)]
        h_m = pltpu.make_async_copy(src, mv_bufs[b], in_sems[b][0])
        h_m.start()
        src = s_hbm.at[pl.ds(slab * in_sz_s, in_sz_s)]
        h_s = pltpu.make_async_copy(src, s_bufs[b], in_sems[b][1])
        h_s.start()
        return h_m, h_s

    out_handles = []
    pending = start_in(0, 0)
    for si in range(n_slabs_w):
        b = si % 2
        slab = wid * n_slabs_w + si
        pending[0].wait()
        pending[1].wait()
        if si + 1 < n_slabs_w:
            pending = start_in(si + 1, (si + 1) % 2)
        if len(out_handles) >= 2:
            for h in out_handles.pop(0):
                h.wait()
        lax.fori_loop(0, CHUNK // LANES,
                      make_tile_body(mv_bufs[b], s_bufs[b],
                                     w_bufs[b], i_bufs[b]), 0)
        hw = pltpu.make_async_copy(
            w_bufs[b], wout_hbm.at[pl.ds(slab * out_sz, out_sz)], out_sems[b])
        hw.start()
        hi = pltpu.make_async_copy(
            i_bufs[b], iout_hbm.at[pl.ds(slab * out_sz, out_sz)], out_sems[b])
        hi.start()
        out_handles.append((hw, hi))
    for hs in out_handles:
        for h in hs:
            h.wait()


def _sc_route(masked_slabs, scores_slabs):
    n_slabs = masked_slabs.shape[0]
    T = n_slabs * CHUNK
    n_slabs_w = n_slabs // N_WORKERS
    mesh = plsc.VectorSubcoreMesh(core_axis_name="c", subcore_axis_name="s")
    return pl.kernel(
        functools.partial(_sc_route_body, n_slabs_w),
        mesh=mesh,
        out_type=[
            jax.ShapeDtypeStruct((T * TOP_K,), jnp.float32),
            jax.ShapeDtypeStruct((T * TOP_K,), jnp.int32),
        ],
        scratch_types=[
            pltpu.VMEM((TOP_K * CHUNK,), jnp.int32),
            pltpu.VMEM((TOP_K * CHUNK,), jnp.int32),
            pltpu.VMEM((E * CHUNK,), jnp.float32),
            pltpu.VMEM((E * CHUNK,), jnp.float32),
            pltpu.VMEM((TOP_K * CHUNK,), jnp.float32),
            pltpu.VMEM((TOP_K * CHUNK,), jnp.float32),
            pltpu.VMEM((TOP_K * CHUNK,), jnp.int32),
            pltpu.VMEM((TOP_K * CHUNK,), jnp.int32),
            pltpu.SemaphoreType.DMA,
            pltpu.SemaphoreType.DMA,
            pltpu.SemaphoreType.DMA,
            pltpu.SemaphoreType.DMA,
            pltpu.SemaphoreType.DMA,
            pltpu.SemaphoreType.DMA,
        ],
        compiler_params=pltpu.CompilerParams(use_tc_tiling_on_sc=False,
                                             needs_layout_passes=False),
    )(masked_slabs.reshape(-1), scores_slabs.reshape(-1))


N_MACRO_CHUNKS = 2


@jax.jit
def kernel(x_TD, kernel_DE, bias_E):
    T = x_TD.shape[0]
    nch = N_MACRO_CHUNKS if T % (N_MACRO_CHUNKS * N_WORKERS * CHUNK) == 0 else 1
    tc = T // nch
    slabs = [_tc_stage(x_TD, kernel_DE, bias_E, chunk_idx=c, n_chunks=nch)
             for c in range(nch)]
    outs = [_sc_route(m, s) for m, s in slabs]
    ws = [wf.reshape(tc // CHUNK, TOP_K, CHUNK).transpose(0, 2, 1)
          for wf, _ in outs]
    idxs = [if_.reshape(tc // CHUNK, TOP_K, CHUNK).transpose(0, 2, 1)
            for _, if_ in outs]
    w = jnp.concatenate(ws, axis=0).reshape(T, TOP_K)
    idx = jnp.concatenate(idxs, axis=0).reshape(T, TOP_K)
    return (w, idx)
